# baseline (speedup 1.0000x reference)
import jax
import jax.numpy as jnp
from jax.experimental import pallas as pl
from jax.experimental.pallas import tpu as pltpu

_N = 2048
_F = 256
_D = 512
_K = 32
_ALPHA = 3.0
_RB = 512
_C = 128
_NC = _N // _C
_S = _N // _RB


def _tri_consts():
    li = jax.lax.broadcasted_iota(jnp.int32, (_C, _C), 0)
    lj = jax.lax.broadcasted_iota(jnp.int32, (_C, _C), 1)
    u_incl = jnp.where(li <= lj, 1.0, 0.0).astype(jnp.bfloat16)
    ci = jax.lax.broadcasted_iota(jnp.int32, (_NC, _NC), 0)
    cj = jax.lax.broadcasted_iota(jnp.int32, (_NC, _NC), 1)
    u_strict = jnp.where(ci < cj, 1.0, 0.0).astype(jnp.bfloat16)
    return u_incl, u_strict


def _chunk(j):
    return slice(j * _C, (j + 1) * _C)


def _row_max(ref):
    cms = [jnp.max(ref[:, _chunk(j)], axis=1, keepdims=True)
           for j in range(_NC)]
    return jnp.max(jnp.concatenate(cms, axis=1), axis=1, keepdims=True)


def _scan_counts(pjb_s):
    _, u_strict = _tri_consts()
    dn = (((1,), (0,)), ((), ()))
    ct = jnp.concatenate(
        [pjb_s[:, j * _C + _C - 1:j * _C + _C].astype(jnp.float32)
         for j in range(_NC)], axis=1)
    cpc = jax.lax.dot_general(ct.astype(jnp.bfloat16), u_strict, dn,
                              preferred_element_type=jnp.float32)
    cnt = cpc[:, _NC - 1:_NC] + ct[:, _NC - 1:_NC]
    return cpc, cnt


def _graph_kernel(x_ref, w1_ref, b1_ref, w2_ref, b2_ref, out_ref,
                  nvc1_s, nvc2_s, eqf_s, pjb_s, vbuf, rem_ref):
    i = pl.program_id(1)
    u_incl, _ = _tri_consts()
    dn = (((1,), (0,)), ((), ()))

    @pl.when(i == 0)
    def _():
        x = x_ref[0]
        dnf = (((1,), (1,)), ((), ()))
        h1 = jax.lax.dot_general(x, w1_ref[...], dnf,
                                 preferred_element_type=jnp.float32)
        t1 = jnp.tanh(_ALPHA * (h1 + b1_ref[...]))
        h2 = jax.lax.dot_general(x, w2_ref[...], dnf,
                                 preferred_element_type=jnp.float32)
        t2 = jnp.tanh(_ALPHA * (h2 + b2_ref[...]))
        nvc1_s[:, :_D] = t1
        nvc1_s[:, _D:] = t2
        nvc2_s[:, :_D] = t2
        nvc2_s[:, _D:] = -t1

    r0 = i * _RB
    lhs = nvc1_s[pl.ds(r0, _RB), :]
    rhs = nvc2_s[...]
    dnd = (((1,), (1,)), ((), ()))
    a = jax.lax.dot_general(lhs, rhs, dnd, preferred_element_type=jnp.float32)
    eqf_s[...] = jnp.where(jnp.tanh(_ALPHA * a) == 1.0, 1.0,
                           0.0).astype(jnp.bfloat16)

    for j in range(_NC):
        pj = jax.lax.dot_general(eqf_s[:, _chunk(j)], u_incl, dn,
                                 preferred_element_type=jnp.float32)
        pjb_s[:, _chunk(j)] = pj.astype(jnp.bfloat16)
    bi = jax.lax.broadcasted_iota(jnp.int32, (_N, _NC), 0)
    bj = jax.lax.broadcasted_iota(jnp.int32, (_N, _NC), 1)
    bmat = jnp.where(bi // _C == bj, 1.0, 0.0).astype(jnp.bfloat16)
    ct = jax.lax.dot_general(eqf_s[...], bmat, dn,
                             preferred_element_type=jnp.float32)
    _, u_strict = _tri_consts()
    cpc = jax.lax.dot_general(ct.astype(jnp.bfloat16), u_strict, dn,
                              preferred_element_type=jnp.float32)
    cnt = cpc[:, _NC - 1:_NC] + ct[:, _NC - 1:_NC]
    fast = jnp.min(cnt) >= float(_K)

    @pl.when(fast)
    def _():
        zb = jnp.zeros((_RB, _C), jnp.bfloat16)
        for j in range(_NC):
            sl = _chunk(j)
            thr = jnp.clip(float(_K) - cpc[:, j:j + 1], -1.0,
                           127.0).astype(jnp.bfloat16)
            out_ref[0, :, sl] = jnp.where(pjb_s[:, sl] <= thr,
                                          eqf_s[:, sl],
                                          zb).astype(jnp.float32)

    @pl.when(jnp.logical_not(fast))
    def _():
        a2 = jax.lax.dot_general(lhs, rhs, dnd,
                                 preferred_element_type=jnp.float32)
        vbuf[...] = jnp.maximum(jnp.tanh(_ALPHA * a2), 0.0)

        def pass_chunks(m):
            for j in range(_NC):
                eqf = jnp.where(vbuf[:, _chunk(j)] == m, 1.0,
                                0.0).astype(jnp.bfloat16)
                pj = jax.lax.dot_general(eqf, u_incl, dn,
                                         preferred_element_type=jnp.float32)
                pjb_s[:, _chunk(j)] = pj.astype(jnp.bfloat16)
            return _scan_counts(pjb_s)

        def take_chunk(j, m, cpc2, rem_f):
            sl = _chunk(j)
            thr = rem_f - cpc2[:, j:j + 1]
            return ((vbuf[:, sl] == m)
                    & (pjb_s[:, sl].astype(jnp.float32) <= thr))

        m1 = _row_max(vbuf)
        cpc1, cnt1 = pass_chunks(m1)
        kf = jnp.full((_RB, 1), float(_K), jnp.float32)
        for j in range(_NC):
            sl = _chunk(j)
            out_ref[0, :, sl] = jnp.where(take_chunk(j, m1, cpc1, kf),
                                          vbuf[:, sl], 0.0)
        rem1 = _K - jnp.minimum(cnt1.astype(jnp.int32), _K)
        tot1 = jnp.sum(rem1)

        @pl.when(tot1 > 0)
        def _():
            for j in range(_NC):
                sl = _chunk(j)
                vbuf[:, sl] = jnp.where(take_chunk(j, m1, cpc1, kf),
                                        -1.0, vbuf[:, sl])
            rem_ref[...] = rem1

        def cond(carry):
            tot, it = carry
            return (tot > 0) & (it < _K)

        def body(carry):
            tot, it = carry
            remv = rem_ref[...]
            rem_f = remv.astype(jnp.float32)
            m2 = _row_max(vbuf)
            cpc2, cnt2 = pass_chunks(m2)
            for j in range(_NC):
                sl = _chunk(j)
                out_ref[0, :, sl] = jnp.where(take_chunk(j, m2, cpc2, rem_f),
                                              vbuf[:, sl], out_ref[0, :, sl])
            rem_new = remv - jnp.minimum(cnt2.astype(jnp.int32), remv)
            tot_new = jnp.sum(rem_new)

            @pl.when(tot_new > 0)
            def _():
                for j in range(_NC):
                    sl = _chunk(j)
                    vbuf[:, sl] = jnp.where(
                        take_chunk(j, m2, cpc2, rem_f), -1.0, vbuf[:, sl])
                rem_ref[...] = rem_new

            return (tot_new, it + 1)

        jax.lax.while_loop(cond, body, (tot1, jnp.int32(0)))


def kernel(X, W1, b1, W2, b2):
    B = X.shape[0]
    b1r = b1.reshape(1, _D)
    b2r = b2.reshape(1, _D)

    adj = pl.pallas_call(
        _graph_kernel,
        grid=(B, _S),
        in_specs=[
            pl.BlockSpec((1, _N, _F), lambda b, i: (b, 0, 0)),
            pl.BlockSpec((_D, _F), lambda b, i: (0, 0)),
            pl.BlockSpec((1, _D), lambda b, i: (0, 0)),
            pl.BlockSpec((_D, _F), lambda b, i: (0, 0)),
            pl.BlockSpec((1, _D), lambda b, i: (0, 0)),
        ],
        out_specs=pl.BlockSpec((1, _RB, _N), lambda b, i: (b, i, 0)),
        out_shape=jax.ShapeDtypeStruct((B, _N, _N), jnp.float32),
        scratch_shapes=[
            pltpu.VMEM((_N, 2 * _D), jnp.float32),
            pltpu.VMEM((_N, 2 * _D), jnp.float32),
            pltpu.VMEM((_RB, _N), jnp.bfloat16),
            pltpu.VMEM((_RB, _N), jnp.bfloat16),
            pltpu.VMEM((_RB, _N), jnp.float32),
            pltpu.VMEM((_RB, 1), jnp.int32),
        ],
        compiler_params=pltpu.CompilerParams(
            vmem_limit_bytes=63 * 1024 * 1024),
    )(X, W1, b1r, W2, b2r)

    return adj


# speedup vs baseline: 1.1200x; 1.1200x over previous
import jax
import jax.numpy as jnp
from jax.experimental import pallas as pl
from jax.experimental.pallas import tpu as pltpu

_N = 2048
_F = 256
_D = 512
_K = 32
_ALPHA = 3.0
_RB = 512
_C = 128
_NC = _N // _C
_S = _N // _RB


def _tri_consts():
    li = jax.lax.broadcasted_iota(jnp.int32, (_C, _C), 0)
    lj = jax.lax.broadcasted_iota(jnp.int32, (_C, _C), 1)
    u_incl = jnp.where(li <= lj, 1.0, 0.0).astype(jnp.bfloat16)
    ci = jax.lax.broadcasted_iota(jnp.int32, (_NC, _NC), 0)
    cj = jax.lax.broadcasted_iota(jnp.int32, (_NC, _NC), 1)
    u_strict = jnp.where(ci < cj, 1.0, 0.0).astype(jnp.bfloat16)
    return u_incl, u_strict


def _chunk(j):
    return slice(j * _C, (j + 1) * _C)


def _row_max(ref):
    cms = [jnp.max(ref[:, _chunk(j)], axis=1, keepdims=True)
           for j in range(_NC)]
    return jnp.max(jnp.concatenate(cms, axis=1), axis=1, keepdims=True)


def _scan_counts(pjb_s):
    _, u_strict = _tri_consts()
    dn = (((1,), (0,)), ((), ()))
    ct = jnp.concatenate(
        [pjb_s[:, j * _C + _C - 1:j * _C + _C].astype(jnp.float32)
         for j in range(_NC)], axis=1)
    cpc = jax.lax.dot_general(ct.astype(jnp.bfloat16), u_strict, dn,
                              preferred_element_type=jnp.float32)
    cnt = cpc[:, _NC - 1:_NC] + ct[:, _NC - 1:_NC]
    return cpc, cnt


def _graph_kernel(x_ref, w1_ref, b1_ref, w2_ref, b2_ref, out_ref,
                  nvc1_s, nvc2_s, eqf_s, pjb_s, vbuf, rem_ref):
    i = pl.program_id(1)
    u_incl, _ = _tri_consts()
    dn = (((1,), (0,)), ((), ()))

    @pl.when(i == 0)
    def _():
        x = x_ref[0]
        dnf = (((1,), (1,)), ((), ()))
        h1 = jax.lax.dot_general(x, w1_ref[...], dnf,
                                 preferred_element_type=jnp.float32)
        t1 = jnp.tanh(_ALPHA * (h1 + b1_ref[...]))
        h2 = jax.lax.dot_general(x, w2_ref[...], dnf,
                                 preferred_element_type=jnp.float32)
        t2 = jnp.tanh(_ALPHA * (h2 + b2_ref[...]))
        nvc1_s[:, :_D] = t1
        nvc1_s[:, _D:] = t2
        nvc2_s[:, :_D] = t2
        nvc2_s[:, _D:] = -t1

    r0 = i * _RB
    lhs = nvc1_s[pl.ds(r0, _RB), :]
    rhs = nvc2_s[...]
    dnd = (((1,), (1,)), ((), ()))
    a = jax.lax.dot_general(lhs, rhs, dnd, preferred_element_type=jnp.float32)
    eqf_s[...] = jnp.where(jnp.tanh(_ALPHA * a) == 1.0, 1.0,
                           0.0).astype(jnp.bfloat16)

    for j in range(_NC):
        pj = jax.lax.dot_general(eqf_s[:, _chunk(j)], u_incl, dn,
                                 preferred_element_type=jnp.float32)
        pjb_s[:, _chunk(j)] = pj.astype(jnp.bfloat16)
    cpc, cnt = _scan_counts(pjb_s)
    fast = jnp.min(cnt) >= float(_K)

    @pl.when(fast)
    def _():
        zb = jnp.zeros((_RB, _C), jnp.bfloat16)
        for j in range(_NC):
            sl = _chunk(j)
            thr = jnp.clip(float(_K) - cpc[:, j:j + 1], -1.0,
                           127.0).astype(jnp.bfloat16)
            out_ref[0, :, sl] = jnp.where(pjb_s[:, sl] <= thr,
                                          eqf_s[:, sl],
                                          zb).astype(jnp.float32)

    @pl.when(jnp.logical_not(fast))
    def _():
        a2 = jax.lax.dot_general(lhs, rhs, dnd,
                                 preferred_element_type=jnp.float32)
        vbuf[...] = jnp.maximum(jnp.tanh(_ALPHA * a2), 0.0)

        def pass_chunks(m):
            for j in range(_NC):
                eqf = jnp.where(vbuf[:, _chunk(j)] == m, 1.0,
                                0.0).astype(jnp.bfloat16)
                pj = jax.lax.dot_general(eqf, u_incl, dn,
                                         preferred_element_type=jnp.float32)
                pjb_s[:, _chunk(j)] = pj.astype(jnp.bfloat16)
            return _scan_counts(pjb_s)

        def take_chunk(j, m, cpc2, rem_f):
            sl = _chunk(j)
            thr = rem_f - cpc2[:, j:j + 1]
            return ((vbuf[:, sl] == m)
                    & (pjb_s[:, sl].astype(jnp.float32) <= thr))

        m1 = _row_max(vbuf)
        cpc1, cnt1 = pass_chunks(m1)
        kf = jnp.full((_RB, 1), float(_K), jnp.float32)
        for j in range(_NC):
            sl = _chunk(j)
            out_ref[0, :, sl] = jnp.where(take_chunk(j, m1, cpc1, kf),
                                          vbuf[:, sl], 0.0)
        rem1 = _K - jnp.minimum(cnt1.astype(jnp.int32), _K)
        tot1 = jnp.sum(rem1)

        @pl.when(tot1 > 0)
        def _():
            for j in range(_NC):
                sl = _chunk(j)
                vbuf[:, sl] = jnp.where(take_chunk(j, m1, cpc1, kf),
                                        -1.0, vbuf[:, sl])
            rem_ref[...] = rem1

        def cond(carry):
            tot, it = carry
            return (tot > 0) & (it < _K)

        def body(carry):
            tot, it = carry
            remv = rem_ref[...]
            rem_f = remv.astype(jnp.float32)
            m2 = _row_max(vbuf)
            cpc2, cnt2 = pass_chunks(m2)
            for j in range(_NC):
                sl = _chunk(j)
                out_ref[0, :, sl] = jnp.where(take_chunk(j, m2, cpc2, rem_f),
                                              vbuf[:, sl], out_ref[0, :, sl])
            rem_new = remv - jnp.minimum(cnt2.astype(jnp.int32), remv)
            tot_new = jnp.sum(rem_new)

            @pl.when(tot_new > 0)
            def _():
                for j in range(_NC):
                    sl = _chunk(j)
                    vbuf[:, sl] = jnp.where(
                        take_chunk(j, m2, cpc2, rem_f), -1.0, vbuf[:, sl])
                rem_ref[...] = rem_new

            return (tot_new, it + 1)

        jax.lax.while_loop(cond, body, (tot1, jnp.int32(0)))


def kernel(X, W1, b1, W2, b2):
    B = X.shape[0]
    b1r = b1.reshape(1, _D)
    b2r = b2.reshape(1, _D)

    adj = pl.pallas_call(
        _graph_kernel,
        grid=(B, _S),
        in_specs=[
            pl.BlockSpec((1, _N, _F), lambda b, i: (b, 0, 0)),
            pl.BlockSpec((_D, _F), lambda b, i: (0, 0)),
            pl.BlockSpec((1, _D), lambda b, i: (0, 0)),
            pl.BlockSpec((_D, _F), lambda b, i: (0, 0)),
            pl.BlockSpec((1, _D), lambda b, i: (0, 0)),
        ],
        out_specs=pl.BlockSpec((1, _RB, _N), lambda b, i: (b, i, 0)),
        out_shape=jax.ShapeDtypeStruct((B, _N, _N), jnp.float32),
        scratch_shapes=[
            pltpu.VMEM((_N, 2 * _D), jnp.float32),
            pltpu.VMEM((_N, 2 * _D), jnp.float32),
            pltpu.VMEM((_RB, _N), jnp.bfloat16),
            pltpu.VMEM((_RB, _N), jnp.bfloat16),
            pltpu.VMEM((_RB, _N), jnp.float32),
            pltpu.VMEM((_RB, 1), jnp.int32),
        ],
        compiler_params=pltpu.CompilerParams(
            vmem_limit_bytes=63 * 1024 * 1024),
    )(X, W1, b1r, W2, b2r)

    return adj


# speedup vs baseline: 1.1849x; 1.0579x over previous
import jax
import jax.numpy as jnp
from jax.experimental import pallas as pl
from jax.experimental.pallas import tpu as pltpu

_N = 2048
_F = 256
_D = 512
_K = 32
_ALPHA = 3.0
_RB = 1024
_C = 128
_NC = _N // _C
_S = _N // _RB


def _tri_consts():
    li = jax.lax.broadcasted_iota(jnp.int32, (_C, _C), 0)
    lj = jax.lax.broadcasted_iota(jnp.int32, (_C, _C), 1)
    u_incl = jnp.where(li <= lj, 1.0, 0.0).astype(jnp.bfloat16)
    ci = jax.lax.broadcasted_iota(jnp.int32, (_NC, _NC), 0)
    cj = jax.lax.broadcasted_iota(jnp.int32, (_NC, _NC), 1)
    u_strict = jnp.where(ci < cj, 1.0, 0.0).astype(jnp.bfloat16)
    return u_incl, u_strict


def _chunk(j):
    return slice(j * _C, (j + 1) * _C)


def _row_max(ref):
    cms = [jnp.max(ref[:, _chunk(j)], axis=1, keepdims=True)
           for j in range(_NC)]
    return jnp.max(jnp.concatenate(cms, axis=1), axis=1, keepdims=True)


def _scan_counts(pjb_s):
    _, u_strict = _tri_consts()
    dn = (((1,), (0,)), ((), ()))
    ct = jnp.concatenate(
        [pjb_s[:, j * _C + _C - 1:j * _C + _C].astype(jnp.float32)
         for j in range(_NC)], axis=1)
    cpc = jax.lax.dot_general(ct.astype(jnp.bfloat16), u_strict, dn,
                              preferred_element_type=jnp.float32)
    cnt = cpc[:, _NC - 1:_NC] + ct[:, _NC - 1:_NC]
    return cpc, cnt


def _graph_kernel(x_ref, w1_ref, b1_ref, w2_ref, b2_ref, out_ref,
                  nvc_s, eqf_s, pjb_s, vbuf):
    i = pl.program_id(1)
    u_incl, _ = _tri_consts()
    dn = (((1,), (0,)), ((), ()))

    @pl.when(i == 0)
    def _():
        x = x_ref[0]
        dnf = (((1,), (1,)), ((), ()))
        h1 = jax.lax.dot_general(x, w1_ref[...], dnf,
                                 preferred_element_type=jnp.float32)
        t1 = jnp.tanh(_ALPHA * (h1 + b1_ref[...]))
        h2 = jax.lax.dot_general(x, w2_ref[...], dnf,
                                 preferred_element_type=jnp.float32)
        t2 = jnp.tanh(_ALPHA * (h2 + b2_ref[...]))
        nvc_s[:, :_D] = t1
        nvc_s[:, _D:] = t2

    r0 = i * _RB
    dnd = (((1,), (1,)), ((), ()))

    def anti_dot():
        t1s = nvc_s[pl.ds(r0, _RB), 0:_D]
        t2s = nvc_s[pl.ds(r0, _RB), _D:2 * _D]
        t1f = nvc_s[:, 0:_D]
        t2f = nvc_s[:, _D:2 * _D]
        return (jax.lax.dot_general(t1s, t2f, dnd,
                                    preferred_element_type=jnp.float32)
                - jax.lax.dot_general(t2s, t1f, dnd,
                                      preferred_element_type=jnp.float32))

    a = anti_dot()
    eqf_s[...] = jnp.where(jnp.tanh(_ALPHA * a) == 1.0, 1.0,
                           0.0).astype(jnp.bfloat16)

    for j in range(_NC):
        pj = jax.lax.dot_general(eqf_s[:, _chunk(j)], u_incl, dn,
                                 preferred_element_type=jnp.float32)
        pjb_s[:, _chunk(j)] = pj.astype(jnp.bfloat16)
    cpc, cnt = _scan_counts(pjb_s)
    fast = jnp.min(cnt) >= float(_K)

    @pl.when(fast)
    def _():
        zb = jnp.zeros((_RB, _C), jnp.bfloat16)
        for j in range(_NC):
            sl = _chunk(j)
            thr = jnp.clip(float(_K) - cpc[:, j:j + 1], -1.0,
                           127.0).astype(jnp.bfloat16)
            out_ref[0, :, sl] = jnp.where(pjb_s[:, sl] <= thr,
                                          eqf_s[:, sl],
                                          zb).astype(jnp.float32)

    @pl.when(jnp.logical_not(fast))
    def _():
        vbuf[...] = jnp.maximum(jnp.tanh(_ALPHA * anti_dot()), 0.0)
        for j in range(_NC):
            out_ref[0, :, _chunk(j)] = jnp.zeros((_RB, _C), jnp.float32)

        def body(it, carry):
            m = _row_max(vbuf)
            cands = []
            for j in range(_NC):
                ci = jax.lax.broadcasted_iota(jnp.int32, (_RB, _C), 1)
                cj = jnp.where(vbuf[:, _chunk(j)] == m, ci + j * _C, _N)
                cands.append(jnp.min(cj, axis=1, keepdims=True))
            amin = jnp.min(jnp.concatenate(cands, axis=1), axis=1,
                           keepdims=True)
            for j in range(_NC):
                sl = _chunk(j)
                ci = jax.lax.broadcasted_iota(jnp.int32, (_RB, _C), 1)
                sel = ci + j * _C == amin
                out_ref[0, :, sl] = jnp.where(sel, vbuf[:, sl],
                                              out_ref[0, :, sl])
                vbuf[:, sl] = jnp.where(sel, -1.0, vbuf[:, sl])
            return carry

        jax.lax.fori_loop(0, _K, body, 0)


def kernel(X, W1, b1, W2, b2):
    B = X.shape[0]
    b1r = b1.reshape(1, _D)
    b2r = b2.reshape(1, _D)

    adj = pl.pallas_call(
        _graph_kernel,
        grid=(B, _S),
        in_specs=[
            pl.BlockSpec((1, _N, _F), lambda b, i: (b, 0, 0)),
            pl.BlockSpec((_D, _F), lambda b, i: (0, 0)),
            pl.BlockSpec((1, _D), lambda b, i: (0, 0)),
            pl.BlockSpec((_D, _F), lambda b, i: (0, 0)),
            pl.BlockSpec((1, _D), lambda b, i: (0, 0)),
        ],
        out_specs=pl.BlockSpec((1, _RB, _N), lambda b, i: (b, i, 0)),
        out_shape=jax.ShapeDtypeStruct((B, _N, _N), jnp.float32),
        scratch_shapes=[
            pltpu.VMEM((_N, 2 * _D), jnp.float32),
            pltpu.VMEM((_RB, _N), jnp.bfloat16),
            pltpu.VMEM((_RB, _N), jnp.bfloat16),
            pltpu.VMEM((_RB, _N), jnp.float32),
        ],
        compiler_params=pltpu.CompilerParams(
            vmem_limit_bytes=63 * 1024 * 1024),
    )(X, W1, b1r, W2, b2r)

    return adj
